# baseline (speedup 1.0000x reference)
import functools

import jax
import jax.numpy as jnp
from jax import lax
from jax.experimental import pallas as pl
from jax.experimental.pallas import tpu as pltpu
from jax.experimental.pallas import tpu_sc as plsc

N = 10000
E = 320000
F_IN = 16
F_OUT = 16
EPS = 1e-5

TE = 8000
GRID = E // TE
TEB = 16000
GRIDB = E // TEB

NC = 2
NS = 16
NW = NC * NS
PER_W = E // NW
SCCH = 2000
NCHUNK = PER_W // SCCH

_ARB = pltpu.CompilerParams(dimension_semantics=("arbitrary",))


def _colsum(z):
    ones_row = jnp.ones((1, z.shape[0]), jnp.float32)
    return jnp.dot(ones_row, z, preferred_element_type=jnp.float32)


def _acc_stats(i, z, s_ref, q_ref):
    ps = _colsum(z)
    pq = _colsum(z * z)

    @pl.when(i == 0)
    def _():
        s_ref[...] = ps
        q_ref[...] = pq

    @pl.when(i > 0)
    def _():
        s_ref[...] = s_ref[...] + ps
        q_ref[...] = q_ref[...] + pq


def _bn_relu(z, s_ref, q_ref, g_ref, be_ref):
    m = s_ref[...] * (1.0 / E)
    v = q_ref[...] * (1.0 / E) - m * m
    a = g_ref[...] * lax.rsqrt(v + EPS)
    c = be_ref[...] - m * a
    return jnp.maximum(z * a + c, 0.0)


def _lin(h, wt_ref, b_ref):
    return jnp.dot(h, wt_ref[...], preferred_element_type=jnp.float32) + b_ref[...]


def _lin16(h, wt_ref, b_ref):
    return jnp.dot(h.astype(jnp.bfloat16), wt_ref[...],
                   preferred_element_type=jnp.float32) + b_ref[...]


def _h0_of(ef_ref, w0t_ref, b0_ref, ms_ref, m2_ref, g0_ref, be0_ref):
    z0 = _lin(ef_ref[...], w0t_ref, b0_ref)
    inv_e = 1.0 / E
    mu = [ms_ref[0, j] * inv_e for j in range(3)]
    wrow = [w0t_ref[j:j + 1, :] for j in range(3)]
    m0 = mu[0] * wrow[0] + mu[1] * wrow[1] + mu[2] * wrow[2] + b0_ref[...]
    var = jnp.zeros_like(m0)
    for j in range(3):
        for k in range(3):
            cjk = m2_ref[j, k] * inv_e - mu[j] * mu[k]
            var = var + cjk * (wrow[j] * wrow[k])
    a0 = g0_ref[...] * lax.rsqrt(var + EPS)
    c0 = be0_ref[...] - m0 * a0
    return jnp.maximum(z0 * a0 + c0, 0.0)


_MROWS = (E * 3) // 128


def _moments_body(f_ref, ms_ref, m2_ref):
    f = f_ref[...]
    r = lax.broadcasted_iota(jnp.int32, (_MROWS, 128), 0)
    l = lax.broadcasted_iota(jnp.int32, (_MROWS, 128), 1)
    ph = (2 * r + l) % 3
    t = [jnp.where(ph == k, f, 0.0) for k in range(3)]
    zcol = jnp.zeros((1, 1), jnp.float32)
    carry1 = jnp.concatenate([f[1:, 0:1], zcol], axis=0)
    fs1 = jnp.concatenate([f[:, 1:], carry1], axis=1)
    carry2 = jnp.concatenate([f[1:, 0:2], jnp.zeros((1, 2), jnp.float32)],
                             axis=0)
    fs2 = jnp.concatenate([f[:, 2:], carry2], axis=1)

    def tot(x):
        return jnp.sum(x).reshape(1, 1)

    s0, s1v, s2v = tot(t[0]), tot(t[1]), tot(t[2])
    d0, d1, d2 = tot(t[0] * f), tot(t[1] * f), tot(t[2] * f)
    o01 = tot(t[0] * fs1)
    o12 = tot(t[1] * fs1)
    o02 = tot(t[0] * fs2)
    ms_ref[...] = jnp.concatenate([s0, s1v, s2v], axis=1)
    m2_ref[...] = jnp.concatenate(
        [jnp.concatenate([d0, o01, o02], axis=1),
         jnp.concatenate([o01, d1, o12], axis=1),
         jnp.concatenate([o02, o12, d2], axis=1)], axis=0)


def _p1_body(ef_ref, w0t_ref, b0_ref, ms_ref, m2_ref, g0_ref, be0_ref,
             w1t_ref, b1_ref,
             z1_ref, s1_ref, q1_ref):
    i = pl.program_id(0)
    h0 = _h0_of(ef_ref, w0t_ref, b0_ref, ms_ref, m2_ref, g0_ref, be0_ref)
    z1 = _lin16(h0, w1t_ref, b1_ref)
    z1_ref[...] = z1.astype(jnp.bfloat16)
    _acc_stats(i, z1, s1_ref, q1_ref)


def _mid_body(z_ref, s_ref, q_ref, g_ref, be_ref, wt_ref, b_ref,
              zo_ref, so_ref, qo_ref):
    i = pl.program_id(0)
    h = _bn_relu(z_ref[...].astype(jnp.float32), s_ref, q_ref, g_ref, be_ref)
    z = _lin16(h, wt_ref, b_ref)
    zo_ref[...] = z.astype(jnp.bfloat16)
    _acc_stats(i, z, so_ref, qo_ref)


def _p4_body(z3_ref, s3_ref, q3_ref, g3_ref, be3_ref,
             wft_ref, bf_ref, x_ref,
             msg_ref):
    h3 = _bn_relu(z3_ref[...].astype(jnp.float32),
                  s3_ref, q3_ref, g3_ref, be3_ref)
    wv = jax.nn.sigmoid(_lin16(h3, wft_ref, bf_ref))
    x = x_ref[...]
    li = lax.broadcasted_iota(jnp.int32, (F_IN, F_IN * F_OUT), 1)
    ri = lax.broadcasted_iota(jnp.int32, (F_IN, F_IN * F_OUT), 0)
    rmat = (li // F_OUT == ri).astype(jnp.float32)
    si = lax.broadcasted_iota(jnp.int32, (F_IN * F_OUT, F_OUT), 0)
    oi = lax.broadcasted_iota(jnp.int32, (F_IN * F_OUT, F_OUT), 1)
    smat = (si % F_OUT == oi).astype(jnp.float32)
    xrep = jnp.dot(x, rmat, preferred_element_type=jnp.float32)
    msg_ref[...] = jnp.dot(xrep * wv, smat,
                           preferred_element_type=jnp.float32)


def _epi_body(p_ref, d_ref, bias_ref, out_ref):
    p = p_ref[...]
    d = d_ref[...]
    deg = jnp.maximum(d[0] + d[1], 1.0)
    out_ref[...] = (p[0] + p[1]) / deg + bias_ref[...]


def _row(x):
    return x.reshape(1, -1)


def _const_spec(x):
    shape = x.shape
    return pl.BlockSpec(shape, lambda i: (0,) * len(shape))


_EF_SPEC = pl.BlockSpec((TE, 3), lambda i: (i, 0))


def _moments(efeat):
    return pl.pallas_call(
        _moments_body,
        out_shape=[jax.ShapeDtypeStruct((1, 3), jnp.float32),
                   jax.ShapeDtypeStruct((3, 3), jnp.float32)],
    )(efeat.reshape(_MROWS, 128))


def _layer1(efeat, consts):
    return pl.pallas_call(
        _p1_body,
        grid=(GRIDB,),
        in_specs=[pl.BlockSpec((TEB, 3), lambda i: (i, 0))]
        + [_const_spec(c) for c in consts],
        out_specs=[pl.BlockSpec((TEB, 128), lambda i: (i, 0)),
                   pl.BlockSpec((1, 128), lambda i: (0, 0)),
                   pl.BlockSpec((1, 128), lambda i: (0, 0))],
        out_shape=[jax.ShapeDtypeStruct((E, 128), jnp.bfloat16),
                   jax.ShapeDtypeStruct((1, 128), jnp.float32),
                   jax.ShapeDtypeStruct((1, 128), jnp.float32)],
        compiler_params=_ARB,
    )(efeat, *consts)


def _mid(z, consts, cin, cout):
    return pl.pallas_call(
        _mid_body,
        grid=(GRIDB,),
        in_specs=[pl.BlockSpec((TEB, cin), lambda i: (i, 0))]
        + [_const_spec(c) for c in consts],
        out_specs=[pl.BlockSpec((TEB, cout), lambda i: (i, 0)),
                   pl.BlockSpec((1, cout), lambda i: (0, 0)),
                   pl.BlockSpec((1, cout), lambda i: (0, 0))],
        out_shape=[jax.ShapeDtypeStruct((E, cout), jnp.bfloat16),
                   jax.ShapeDtypeStruct((1, cout), jnp.float32),
                   jax.ShapeDtypeStruct((1, cout), jnp.float32)],
        compiler_params=_ARB,
    )(z, *consts)


def _final_call(z3, consts, x_src):
    return pl.pallas_call(
        _p4_body,
        grid=(GRID,),
        in_specs=[pl.BlockSpec((TE, 32), lambda i: (i, 0))]
        + [_const_spec(c) for c in consts]
        + [pl.BlockSpec((TE, F_IN), lambda i: (i, 0))],
        out_specs=pl.BlockSpec((TE, F_OUT), lambda i: (i, 0)),
        out_shape=jax.ShapeDtypeStruct((E, F_OUT), jnp.float32),
        compiler_params=_ARB,
    )(z3, *consts, x_src)


def _epilogue(p2, d2, bias8):
    return pl.pallas_call(
        _epi_body,
        out_shape=jax.ShapeDtypeStruct((N // 8, 128), jnp.float32),
    )(p2, d2, bias8)


@functools.lru_cache(maxsize=None)
def _sc_kernels():
    mesh = plsc.VectorSubcoreMesh(core_axis_name="c", subcore_axis_name="s")

    @functools.partial(
        pl.kernel,
        mesh=mesh,
        out_type=jax.ShapeDtypeStruct((E, F_IN), jnp.float32),
        scratch_types=[pltpu.VMEM((SCCH,), jnp.int32),
                       pltpu.VMEM((SCCH, F_IN), jnp.float32),
                       pltpu.SemaphoreType.DMA],
        compiler_params=pltpu.CompilerParams(use_tc_tiling_on_sc=False),
    )
    def gather(nfeat_hbm, src_hbm, out_hbm, idx_v, rows_v, sem):
        wid = lax.axis_index("s") * NC + lax.axis_index("c")
        base = wid * PER_W

        def body(j, carry):
            off = base + j * SCCH
            pltpu.sync_copy(src_hbm.at[pl.ds(off, SCCH)], idx_v)
            pltpu.async_copy(nfeat_hbm.at[idx_v], rows_v, sem).wait()
            pltpu.sync_copy(rows_v, out_hbm.at[pl.ds(off, SCCH)])
            return carry

        lax.fori_loop(0, NCHUNK, body, 0)

    @functools.partial(
        pl.kernel,
        mesh=mesh,
        out_type=[jax.ShapeDtypeStruct((NC, N, F_OUT), jnp.float32),
                  jax.ShapeDtypeStruct((NC, N, F_OUT), jnp.float32)],
        scratch_types=[pltpu.VMEM((SCCH,), jnp.int32),
                       pltpu.VMEM((SCCH, F_OUT), jnp.float32),
                       pltpu.VMEM((SCCH, F_OUT), jnp.float32),
                       pltpu.VMEM_SHARED((N, F_OUT), jnp.float32),
                       pltpu.VMEM_SHARED((N, F_OUT), jnp.float32)],
        compiler_params=pltpu.CompilerParams(use_tc_tiling_on_sc=False),
    )
    def scatter(msg_hbm, dst_hbm, zeros_hbm, ones_hbm,
                neigh_out, deg_out, idx_v, m_v, ones_v, acc_sh, deg_sh):
        c = lax.axis_index("c")
        s = lax.axis_index("s")
        wid = s * NC + c
        base = wid * PER_W

        @pl.when(s == 0)
        def _():
            pltpu.sync_copy(zeros_hbm, acc_sh)
            pltpu.sync_copy(zeros_hbm, deg_sh)

        pltpu.sync_copy(ones_hbm, ones_v)
        plsc.subcore_barrier()

        def body(j, carry):
            off = base + j * SCCH
            pltpu.sync_copy(dst_hbm.at[pl.ds(off, SCCH)], idx_v)
            pltpu.sync_copy(msg_hbm.at[pl.ds(off, SCCH)], m_v)
            pltpu.sync_copy(m_v, acc_sh.at[idx_v], add=True)
            pltpu.sync_copy(ones_v, deg_sh.at[idx_v], add=True)
            return carry

        lax.fori_loop(0, NCHUNK, body, 0)
        plsc.subcore_barrier()

        @pl.when(s == 0)
        def _():
            pltpu.sync_copy(acc_sh, neigh_out.at[c])
            pltpu.sync_copy(deg_sh, deg_out.at[c])

    return gather, scatter


def _sc_gather(nfeat, src):
    return _sc_kernels()[0](nfeat, src)


def _sc_scatter(msg, dst, zeros, ones):
    return _sc_kernels()[1](msg, dst, zeros, ones)


def kernel(nfeat, edge_index, efeat,
           W0, b0, g0, be0,
           W1, b1, g1, be1,
           W2, b2, g2, be2,
           W3, b3, g3, be3,
           Wf, bf, bias):
    src = edge_index[0]
    dst = edge_index[1]

    ms, m2 = _moments(efeat)
    x_src = _sc_gather(nfeat, src)

    bf16 = jnp.bfloat16
    c1 = [W0.T, _row(b0), ms, m2, _row(g0), _row(be0),
          W1.T.astype(bf16), _row(b1)]
    z1, s1, q1 = _layer1(efeat, c1)
    z2, s2, q2 = _mid(z1, [s1, q1, _row(g1), _row(be1),
                           W2.T.astype(bf16), _row(b2)], 128, 64)
    z3, s3, q3 = _mid(z2, [s2, q2, _row(g2), _row(be2),
                           W3.T.astype(bf16), _row(b3)], 64, 32)
    msg = _final_call(z3, [s3, q3, _row(g3), _row(be3),
                           Wf.T.astype(bf16), _row(bf)], x_src)

    zeros = jnp.zeros((N, F_OUT), jnp.float32)
    ones = jnp.ones((SCCH, F_OUT), jnp.float32)
    part, degp = _sc_scatter(msg, dst, zeros, ones)

    p2 = part.reshape(NC, N // 8, 128)
    d2 = degp.reshape(NC, N // 8, 128)
    bias8 = jnp.tile(bias, 8).reshape(1, 128)
    out = _epilogue(p2, d2, bias8)
    return out.reshape(N, F_OUT)


# speedup vs baseline: 1.0050x; 1.0050x over previous
import functools

import jax
import jax.numpy as jnp
from jax import lax
from jax.experimental import pallas as pl
from jax.experimental.pallas import tpu as pltpu
from jax.experimental.pallas import tpu_sc as plsc

N = 10000
E = 320000
F_IN = 16
F_OUT = 16
EPS = 1e-5

TE = 8000
GRID = E // TE
TEB = 16000
GRIDB = E // TEB

NC = 2
NS = 16
NW = NC * NS
PER_W = E // NW
SCCH = 2000
NCHUNK = PER_W // SCCH

_ARB = pltpu.CompilerParams(dimension_semantics=("arbitrary",))


def _colsum(z):
    ones_row = jnp.ones((1, z.shape[0]), jnp.float32)
    return jnp.dot(ones_row, z, preferred_element_type=jnp.float32)


def _acc_stats(i, z, s_ref, q_ref):
    ps = _colsum(z)
    pq = _colsum(z * z)

    @pl.when(i == 0)
    def _():
        s_ref[...] = ps
        q_ref[...] = pq

    @pl.when(i > 0)
    def _():
        s_ref[...] = s_ref[...] + ps
        q_ref[...] = q_ref[...] + pq


def _bn_relu(z, s_ref, q_ref, g_ref, be_ref):
    m = s_ref[...] * (1.0 / E)
    v = q_ref[...] * (1.0 / E) - m * m
    a = g_ref[...] * lax.rsqrt(v + EPS)
    c = be_ref[...] - m * a
    return jnp.maximum(z * a + c, 0.0)


def _lin(h, wt_ref, b_ref):
    return jnp.dot(h, wt_ref[...], preferred_element_type=jnp.float32) + b_ref[...]


def _lin16(h, wt_ref, b_ref):
    return jnp.dot(h.astype(jnp.bfloat16), wt_ref[...],
                   preferred_element_type=jnp.float32) + b_ref[...]


def _h0_of(ef_ref, w0t_ref, b0_ref, ms_ref, m2_ref, g0_ref, be0_ref):
    z0 = _lin(ef_ref[...], w0t_ref, b0_ref)
    inv_e = 1.0 / E
    mu = [ms_ref[0, j] * inv_e for j in range(3)]
    wrow = [w0t_ref[j:j + 1, :] for j in range(3)]
    m0 = mu[0] * wrow[0] + mu[1] * wrow[1] + mu[2] * wrow[2] + b0_ref[...]
    var = jnp.zeros_like(m0)
    for j in range(3):
        for k in range(3):
            cjk = m2_ref[j, k] * inv_e - mu[j] * mu[k]
            var = var + cjk * (wrow[j] * wrow[k])
    a0 = g0_ref[...] * lax.rsqrt(var + EPS)
    c0 = be0_ref[...] - m0 * a0
    return jnp.maximum(z0 * a0 + c0, 0.0)


_MROWS = (E * 3) // 128


def _moments_body(f_ref, ms_ref, m2_ref):
    f = f_ref[...]
    r = lax.broadcasted_iota(jnp.int32, (_MROWS, 128), 0)
    l = lax.broadcasted_iota(jnp.int32, (_MROWS, 128), 1)
    ph = (2 * r + l) % 3
    t = [jnp.where(ph == k, f, 0.0) for k in range(3)]
    zcol = jnp.zeros((1, 1), jnp.float32)
    carry1 = jnp.concatenate([f[1:, 0:1], zcol], axis=0)
    fs1 = jnp.concatenate([f[:, 1:], carry1], axis=1)
    carry2 = jnp.concatenate([f[1:, 0:2], jnp.zeros((1, 2), jnp.float32)],
                             axis=0)
    fs2 = jnp.concatenate([f[:, 2:], carry2], axis=1)

    def tot(x):
        return jnp.sum(x).reshape(1, 1)

    s0, s1v, s2v = tot(t[0]), tot(t[1]), tot(t[2])
    d0, d1, d2 = tot(t[0] * f), tot(t[1] * f), tot(t[2] * f)
    o01 = tot(t[0] * fs1)
    o12 = tot(t[1] * fs1)
    o02 = tot(t[0] * fs2)
    ms_ref[...] = jnp.concatenate([s0, s1v, s2v], axis=1)
    m2_ref[...] = jnp.concatenate(
        [jnp.concatenate([d0, o01, o02], axis=1),
         jnp.concatenate([o01, d1, o12], axis=1),
         jnp.concatenate([o02, o12, d2], axis=1)], axis=0)


def _p1_body(ef_ref, w0t_ref, b0_ref, ms_ref, m2_ref, g0_ref, be0_ref,
             w1t_ref, b1_ref,
             z1_ref, s1_ref, q1_ref):
    i = pl.program_id(0)
    h0 = _h0_of(ef_ref, w0t_ref, b0_ref, ms_ref, m2_ref, g0_ref, be0_ref)
    z1 = _lin(h0, w1t_ref, b1_ref)
    z1_ref[...] = z1.astype(jnp.bfloat16)
    _acc_stats(i, z1, s1_ref, q1_ref)


def _mid_body(z_ref, s_ref, q_ref, g_ref, be_ref, wt_ref, b_ref,
              zo_ref, so_ref, qo_ref):
    i = pl.program_id(0)
    h = _bn_relu(z_ref[...].astype(jnp.float32), s_ref, q_ref, g_ref, be_ref)
    z = _lin(h, wt_ref, b_ref)
    zo_ref[...] = z.astype(jnp.bfloat16)
    _acc_stats(i, z, so_ref, qo_ref)


def _p4_body(z3_ref, s3_ref, q3_ref, g3_ref, be3_ref,
             wft_ref, bf_ref, x_ref,
             msg_ref):
    h3 = _bn_relu(z3_ref[...].astype(jnp.float32),
                  s3_ref, q3_ref, g3_ref, be3_ref)
    wv = jax.nn.sigmoid(_lin(h3, wft_ref, bf_ref))
    x = x_ref[...]
    li = lax.broadcasted_iota(jnp.int32, (F_IN, F_IN * F_OUT), 1)
    ri = lax.broadcasted_iota(jnp.int32, (F_IN, F_IN * F_OUT), 0)
    rmat = (li // F_OUT == ri).astype(jnp.float32)
    si = lax.broadcasted_iota(jnp.int32, (F_IN * F_OUT, F_OUT), 0)
    oi = lax.broadcasted_iota(jnp.int32, (F_IN * F_OUT, F_OUT), 1)
    smat = (si % F_OUT == oi).astype(jnp.float32)
    xrep = jnp.dot(x, rmat, preferred_element_type=jnp.float32)
    msg_ref[...] = jnp.dot(xrep * wv, smat,
                           preferred_element_type=jnp.float32)


def _epi_body(p_ref, d_ref, bias_ref, out_ref):
    p = p_ref[...]
    d = d_ref[...]
    deg = jnp.maximum(d[0] + d[1], 1.0)
    out_ref[...] = (p[0] + p[1]) / deg + bias_ref[...]


def _row(x):
    return x.reshape(1, -1)


def _const_spec(x):
    shape = x.shape
    return pl.BlockSpec(shape, lambda i: (0,) * len(shape))


_EF_SPEC = pl.BlockSpec((TE, 3), lambda i: (i, 0))


def _moments(efeat):
    return pl.pallas_call(
        _moments_body,
        out_shape=[jax.ShapeDtypeStruct((1, 3), jnp.float32),
                   jax.ShapeDtypeStruct((3, 3), jnp.float32)],
    )(efeat.reshape(_MROWS, 128))


def _layer1(efeat, consts):
    return pl.pallas_call(
        _p1_body,
        grid=(GRIDB,),
        in_specs=[pl.BlockSpec((TEB, 3), lambda i: (i, 0))]
        + [_const_spec(c) for c in consts],
        out_specs=[pl.BlockSpec((TEB, 128), lambda i: (i, 0)),
                   pl.BlockSpec((1, 128), lambda i: (0, 0)),
                   pl.BlockSpec((1, 128), lambda i: (0, 0))],
        out_shape=[jax.ShapeDtypeStruct((E, 128), jnp.bfloat16),
                   jax.ShapeDtypeStruct((1, 128), jnp.float32),
                   jax.ShapeDtypeStruct((1, 128), jnp.float32)],
        compiler_params=_ARB,
    )(efeat, *consts)


def _mid(z, consts, cin, cout):
    return pl.pallas_call(
        _mid_body,
        grid=(GRIDB,),
        in_specs=[pl.BlockSpec((TEB, cin), lambda i: (i, 0))]
        + [_const_spec(c) for c in consts],
        out_specs=[pl.BlockSpec((TEB, cout), lambda i: (i, 0)),
                   pl.BlockSpec((1, cout), lambda i: (0, 0)),
                   pl.BlockSpec((1, cout), lambda i: (0, 0))],
        out_shape=[jax.ShapeDtypeStruct((E, cout), jnp.bfloat16),
                   jax.ShapeDtypeStruct((1, cout), jnp.float32),
                   jax.ShapeDtypeStruct((1, cout), jnp.float32)],
        compiler_params=_ARB,
    )(z, *consts)


def _final_call(z3, consts, x_src):
    return pl.pallas_call(
        _p4_body,
        grid=(GRID,),
        in_specs=[pl.BlockSpec((TE, 32), lambda i: (i, 0))]
        + [_const_spec(c) for c in consts]
        + [pl.BlockSpec((TE, F_IN), lambda i: (i, 0))],
        out_specs=pl.BlockSpec((TE, F_OUT), lambda i: (i, 0)),
        out_shape=jax.ShapeDtypeStruct((E, F_OUT), jnp.float32),
        compiler_params=_ARB,
    )(z3, *consts, x_src)


def _epilogue(p2, d2, bias8):
    return pl.pallas_call(
        _epi_body,
        out_shape=jax.ShapeDtypeStruct((N // 8, 128), jnp.float32),
    )(p2, d2, bias8)


@functools.lru_cache(maxsize=None)
def _sc_kernels():
    mesh = plsc.VectorSubcoreMesh(core_axis_name="c", subcore_axis_name="s")

    @functools.partial(
        pl.kernel,
        mesh=mesh,
        out_type=jax.ShapeDtypeStruct((E, F_IN), jnp.float32),
        scratch_types=[pltpu.VMEM((SCCH,), jnp.int32),
                       pltpu.VMEM((SCCH, F_IN), jnp.float32),
                       pltpu.SemaphoreType.DMA],
        compiler_params=pltpu.CompilerParams(use_tc_tiling_on_sc=False),
    )
    def gather(nfeat_hbm, src_hbm, out_hbm, idx_v, rows_v, sem):
        wid = lax.axis_index("s") * NC + lax.axis_index("c")
        base = wid * PER_W

        def body(j, carry):
            off = base + j * SCCH
            pltpu.sync_copy(src_hbm.at[pl.ds(off, SCCH)], idx_v)
            pltpu.async_copy(nfeat_hbm.at[idx_v], rows_v, sem).wait()
            pltpu.sync_copy(rows_v, out_hbm.at[pl.ds(off, SCCH)])
            return carry

        lax.fori_loop(0, NCHUNK, body, 0)

    @functools.partial(
        pl.kernel,
        mesh=mesh,
        out_type=[jax.ShapeDtypeStruct((NC, N, F_OUT), jnp.float32),
                  jax.ShapeDtypeStruct((NC, N, F_OUT), jnp.float32)],
        scratch_types=[pltpu.VMEM((SCCH,), jnp.int32),
                       pltpu.VMEM((SCCH, F_OUT), jnp.float32),
                       pltpu.VMEM((SCCH, F_OUT), jnp.float32),
                       pltpu.VMEM_SHARED((N, F_OUT), jnp.float32),
                       pltpu.VMEM_SHARED((N, F_OUT), jnp.float32)],
        compiler_params=pltpu.CompilerParams(use_tc_tiling_on_sc=False),
    )
    def scatter(msg_hbm, dst_hbm, zeros_hbm, ones_hbm,
                neigh_out, deg_out, idx_v, m_v, ones_v, acc_sh, deg_sh):
        c = lax.axis_index("c")
        s = lax.axis_index("s")
        wid = s * NC + c
        base = wid * PER_W

        @pl.when(s == 0)
        def _():
            pltpu.sync_copy(zeros_hbm, acc_sh)
            pltpu.sync_copy(zeros_hbm, deg_sh)

        pltpu.sync_copy(ones_hbm, ones_v)
        plsc.subcore_barrier()

        def body(j, carry):
            off = base + j * SCCH
            pltpu.sync_copy(dst_hbm.at[pl.ds(off, SCCH)], idx_v)
            pltpu.sync_copy(msg_hbm.at[pl.ds(off, SCCH)], m_v)
            pltpu.sync_copy(m_v, acc_sh.at[idx_v], add=True)
            pltpu.sync_copy(ones_v, deg_sh.at[idx_v], add=True)
            return carry

        lax.fori_loop(0, NCHUNK, body, 0)
        plsc.subcore_barrier()

        @pl.when(s == 0)
        def _():
            pltpu.sync_copy(acc_sh, neigh_out.at[c])
            pltpu.sync_copy(deg_sh, deg_out.at[c])

    return gather, scatter


def _sc_gather(nfeat, src):
    return _sc_kernels()[0](nfeat, src)


def _sc_scatter(msg, dst, zeros, ones):
    return _sc_kernels()[1](msg, dst, zeros, ones)


def kernel(nfeat, edge_index, efeat,
           W0, b0, g0, be0,
           W1, b1, g1, be1,
           W2, b2, g2, be2,
           W3, b3, g3, be3,
           Wf, bf, bias):
    src = edge_index[0]
    dst = edge_index[1]

    ms, m2 = _moments(efeat)
    x_src = _sc_gather(nfeat, src)

    bf16 = jnp.bfloat16
    c1 = [W0.T, _row(b0), ms, m2, _row(g0), _row(be0),
          W1.T, _row(b1)]
    z1, s1, q1 = _layer1(efeat, c1)
    z2, s2, q2 = _mid(z1, [s1, q1, _row(g1), _row(be1),
                           W2.T, _row(b2)], 128, 64)
    z3, s3, q3 = _mid(z2, [s2, q2, _row(g2), _row(be2),
                           W3.T, _row(b3)], 64, 32)
    msg = _final_call(z3, [s3, q3, _row(g3), _row(be3),
                           Wf.T, _row(bf)], x_src)

    zeros = jnp.zeros((N, F_OUT), jnp.float32)
    ones = jnp.ones((SCCH, F_OUT), jnp.float32)
    part, degp = _sc_scatter(msg, dst, zeros, ones)

    p2 = part.reshape(NC, N // 8, 128)
    d2 = degp.reshape(NC, N // 8, 128)
    bias8 = jnp.tile(bias, 8).reshape(1, 128)
    out = _epilogue(p2, d2, bias8)
    return out.reshape(N, F_OUT)


# speedup vs baseline: 1.2858x; 1.2793x over previous
import functools

import jax
import jax.numpy as jnp
from jax import lax
from jax.experimental import pallas as pl
from jax.experimental.pallas import tpu as pltpu
from jax.experimental.pallas import tpu_sc as plsc

N = 10000
E = 320000
F_IN = 16
F_OUT = 16
EPS = 1e-5

TE = 8000
GRID = E // TE
TEB = 16000
GRIDB = E // TEB

NC = 2
NS = 16
NW = NC * NS
PER_W = E // NW
SCCH = 2000
NCHUNK = PER_W // SCCH

_ARB = pltpu.CompilerParams(dimension_semantics=("arbitrary",))


def _colsum(z):
    ones_row = jnp.ones((1, z.shape[0]), jnp.float32)
    return jnp.dot(ones_row, z, preferred_element_type=jnp.float32)


def _acc_stats(i, z, s_ref, q_ref):
    ps = _colsum(z)
    pq = _colsum(z * z)

    @pl.when(i == 0)
    def _():
        s_ref[...] = ps
        q_ref[...] = pq

    @pl.when(i > 0)
    def _():
        s_ref[...] = s_ref[...] + ps
        q_ref[...] = q_ref[...] + pq


def _bn_relu(z, s_ref, q_ref, g_ref, be_ref):
    m = s_ref[...] * (1.0 / E)
    v = q_ref[...] * (1.0 / E) - m * m
    a = g_ref[...] * lax.rsqrt(v + EPS)
    c = be_ref[...] - m * a
    return jnp.maximum(z * a + c, 0.0)


def _lin(h, wt_ref, b_ref):
    return jnp.dot(h, wt_ref[...], preferred_element_type=jnp.float32) + b_ref[...]


def _lin16(h, wt_ref, b_ref):
    return jnp.dot(h.astype(jnp.bfloat16), wt_ref[...],
                   preferred_element_type=jnp.float32) + b_ref[...]


def _h0_of(ef_ref, w0t_ref, b0_ref, ms_ref, m2_ref, g0_ref, be0_ref):
    z0 = lax.dot_general(ef_ref[...], w0t_ref[...],
                         (((0,), (0,)), ((), ())),
                         preferred_element_type=jnp.float32) + b0_ref[...]
    inv_e = 1.0 / E
    mu = [ms_ref[j, 0] * inv_e for j in range(3)]
    wrow = [w0t_ref[j:j + 1, :] for j in range(3)]
    m0 = mu[0] * wrow[0] + mu[1] * wrow[1] + mu[2] * wrow[2] + b0_ref[...]
    var = jnp.zeros_like(m0)
    for j in range(3):
        for k in range(3):
            cjk = m2_ref[j, k] * inv_e - mu[j] * mu[k]
            var = var + cjk * (wrow[j] * wrow[k])
    a0 = g0_ref[...] * lax.rsqrt(var + EPS)
    c0 = be0_ref[...] - m0 * a0
    return jnp.maximum(z0 * a0 + c0, 0.0)


def _moments_body(f_ref, ms_ref, m2_ref):
    e = f_ref[...]
    ones_col = jnp.ones((1, E), jnp.float32)
    ms_ref[...] = lax.dot_general(e, ones_col, (((1,), (1,)), ((), ())),
                                  preferred_element_type=jnp.float32)
    m2_ref[...] = lax.dot_general(e, e, (((1,), (1,)), ((), ())),
                                  preferred_element_type=jnp.float32)


def _p1_body(ef_ref, w0t_ref, b0_ref, ms_ref, m2_ref, g0_ref, be0_ref,
             w1t_ref, b1_ref,
             z1_ref, s1_ref, q1_ref):
    i = pl.program_id(0)
    h0 = _h0_of(ef_ref, w0t_ref, b0_ref, ms_ref, m2_ref, g0_ref, be0_ref)
    z1 = _lin(h0, w1t_ref, b1_ref)
    z1_ref[...] = z1.astype(jnp.bfloat16)
    _acc_stats(i, z1, s1_ref, q1_ref)


def _mid_body(z_ref, s_ref, q_ref, g_ref, be_ref, wt_ref, b_ref,
              zo_ref, so_ref, qo_ref):
    i = pl.program_id(0)
    h = _bn_relu(z_ref[...].astype(jnp.float32), s_ref, q_ref, g_ref, be_ref)
    z = _lin(h, wt_ref, b_ref)
    zo_ref[...] = z.astype(jnp.bfloat16)
    _acc_stats(i, z, so_ref, qo_ref)


def _p4_body(z3_ref, s3_ref, q3_ref, g3_ref, be3_ref,
             wft_ref, bf_ref, x_ref,
             msg_ref):
    h3 = _bn_relu(z3_ref[...].astype(jnp.float32),
                  s3_ref, q3_ref, g3_ref, be3_ref)
    wv = jax.nn.sigmoid(_lin(h3, wft_ref, bf_ref))
    x = x_ref[...]
    li = lax.broadcasted_iota(jnp.int32, (F_IN, F_IN * F_OUT), 1)
    ri = lax.broadcasted_iota(jnp.int32, (F_IN, F_IN * F_OUT), 0)
    rmat = (li // F_OUT == ri).astype(jnp.float32)
    si = lax.broadcasted_iota(jnp.int32, (F_IN * F_OUT, F_OUT), 0)
    oi = lax.broadcasted_iota(jnp.int32, (F_IN * F_OUT, F_OUT), 1)
    smat = (si % F_OUT == oi).astype(jnp.float32)
    xrep = jnp.dot(x, rmat, preferred_element_type=jnp.float32)
    msg_ref[...] = jnp.dot(xrep * wv, smat,
                           preferred_element_type=jnp.float32)


def _epi_body(p_ref, d_ref, bias_ref, out_ref):
    p = p_ref[...]
    d = d_ref[...]
    deg = jnp.maximum(d[0] + d[1], 1.0)
    out_ref[...] = (p[0] + p[1]) / deg + bias_ref[...]


def _row(x):
    return x.reshape(1, -1)


def _const_spec(x):
    shape = x.shape
    return pl.BlockSpec(shape, lambda i: (0,) * len(shape))


_EF_SPEC = pl.BlockSpec((TE, 3), lambda i: (i, 0))


def _moments(ef_t):
    return pl.pallas_call(
        _moments_body,
        out_shape=[jax.ShapeDtypeStruct((3, 1), jnp.float32),
                   jax.ShapeDtypeStruct((3, 3), jnp.float32)],
    )(ef_t)


def _layer1(ef_t, consts):
    return pl.pallas_call(
        _p1_body,
        grid=(GRIDB,),
        in_specs=[pl.BlockSpec((3, TEB), lambda i: (0, i))]
        + [_const_spec(c) for c in consts],
        out_specs=[pl.BlockSpec((TEB, 128), lambda i: (i, 0)),
                   pl.BlockSpec((1, 128), lambda i: (0, 0)),
                   pl.BlockSpec((1, 128), lambda i: (0, 0))],
        out_shape=[jax.ShapeDtypeStruct((E, 128), jnp.bfloat16),
                   jax.ShapeDtypeStruct((1, 128), jnp.float32),
                   jax.ShapeDtypeStruct((1, 128), jnp.float32)],
        compiler_params=_ARB,
    )(ef_t, *consts)


def _mid(z, consts, cin, cout):
    return pl.pallas_call(
        _mid_body,
        grid=(GRIDB,),
        in_specs=[pl.BlockSpec((TEB, cin), lambda i: (i, 0))]
        + [_const_spec(c) for c in consts],
        out_specs=[pl.BlockSpec((TEB, cout), lambda i: (i, 0)),
                   pl.BlockSpec((1, cout), lambda i: (0, 0)),
                   pl.BlockSpec((1, cout), lambda i: (0, 0))],
        out_shape=[jax.ShapeDtypeStruct((E, cout), jnp.bfloat16),
                   jax.ShapeDtypeStruct((1, cout), jnp.float32),
                   jax.ShapeDtypeStruct((1, cout), jnp.float32)],
        compiler_params=_ARB,
    )(z, *consts)


def _final_call(z3, consts, x_src):
    return pl.pallas_call(
        _p4_body,
        grid=(GRID,),
        in_specs=[pl.BlockSpec((TE, 32), lambda i: (i, 0))]
        + [_const_spec(c) for c in consts]
        + [pl.BlockSpec((TE, F_IN), lambda i: (i, 0))],
        out_specs=pl.BlockSpec((TE, F_OUT), lambda i: (i, 0)),
        out_shape=jax.ShapeDtypeStruct((E, F_OUT), jnp.float32),
        compiler_params=_ARB,
    )(z3, *consts, x_src)


def _epilogue(p2, d2, bias8):
    return pl.pallas_call(
        _epi_body,
        out_shape=jax.ShapeDtypeStruct((N // 8, 128), jnp.float32),
    )(p2, d2, bias8)


@functools.lru_cache(maxsize=None)
def _sc_kernels():
    mesh = plsc.VectorSubcoreMesh(core_axis_name="c", subcore_axis_name="s")

    @functools.partial(
        pl.kernel,
        mesh=mesh,
        out_type=jax.ShapeDtypeStruct((E, F_IN), jnp.float32),
        scratch_types=[pltpu.VMEM((SCCH,), jnp.int32),
                       pltpu.VMEM((SCCH, F_IN), jnp.float32),
                       pltpu.SemaphoreType.DMA],
        compiler_params=pltpu.CompilerParams(use_tc_tiling_on_sc=False),
    )
    def gather(nfeat_hbm, src_hbm, out_hbm, idx_v, rows_v, sem):
        wid = lax.axis_index("s") * NC + lax.axis_index("c")
        base = wid * PER_W

        def body(j, carry):
            off = base + j * SCCH
            pltpu.sync_copy(src_hbm.at[pl.ds(off, SCCH)], idx_v)
            pltpu.async_copy(nfeat_hbm.at[idx_v], rows_v, sem).wait()
            pltpu.sync_copy(rows_v, out_hbm.at[pl.ds(off, SCCH)])
            return carry

        lax.fori_loop(0, NCHUNK, body, 0)

    @functools.partial(
        pl.kernel,
        mesh=mesh,
        out_type=[jax.ShapeDtypeStruct((NC, N, F_OUT), jnp.float32),
                  jax.ShapeDtypeStruct((NC, N, F_OUT), jnp.float32)],
        scratch_types=[pltpu.VMEM((SCCH,), jnp.int32),
                       pltpu.VMEM((SCCH, F_OUT), jnp.float32),
                       pltpu.VMEM((SCCH, F_OUT), jnp.float32),
                       pltpu.VMEM_SHARED((N, F_OUT), jnp.float32),
                       pltpu.VMEM_SHARED((N, F_OUT), jnp.float32)],
        compiler_params=pltpu.CompilerParams(use_tc_tiling_on_sc=False),
    )
    def scatter(msg_hbm, dst_hbm, zeros_hbm, ones_hbm,
                neigh_out, deg_out, idx_v, m_v, ones_v, acc_sh, deg_sh):
        c = lax.axis_index("c")
        s = lax.axis_index("s")
        wid = s * NC + c
        base = wid * PER_W

        @pl.when(s == 0)
        def _():
            pltpu.sync_copy(zeros_hbm, acc_sh)
            pltpu.sync_copy(zeros_hbm, deg_sh)

        pltpu.sync_copy(ones_hbm, ones_v)
        plsc.subcore_barrier()

        def body(j, carry):
            off = base + j * SCCH
            pltpu.sync_copy(dst_hbm.at[pl.ds(off, SCCH)], idx_v)
            pltpu.sync_copy(msg_hbm.at[pl.ds(off, SCCH)], m_v)
            pltpu.sync_copy(m_v, acc_sh.at[idx_v], add=True)
            pltpu.sync_copy(ones_v, deg_sh.at[idx_v], add=True)
            return carry

        lax.fori_loop(0, NCHUNK, body, 0)
        plsc.subcore_barrier()

        @pl.when(s == 0)
        def _():
            pltpu.sync_copy(acc_sh, neigh_out.at[c])
            pltpu.sync_copy(deg_sh, deg_out.at[c])

    return gather, scatter


def _sc_gather(nfeat, src):
    return _sc_kernels()[0](nfeat, src)


def _sc_scatter(msg, dst, zeros, ones):
    return _sc_kernels()[1](msg, dst, zeros, ones)


def kernel(nfeat, edge_index, efeat,
           W0, b0, g0, be0,
           W1, b1, g1, be1,
           W2, b2, g2, be2,
           W3, b3, g3, be3,
           Wf, bf, bias):
    src = edge_index[0]
    dst = edge_index[1]

    ef_t = efeat.T
    ms, m2 = _moments(ef_t)
    x_src = _sc_gather(nfeat, src)

    c1 = [W0.T, _row(b0), ms, m2, _row(g0), _row(be0),
          W1.T, _row(b1)]
    z1, s1, q1 = _layer1(ef_t, c1)
    z2, s2, q2 = _mid(z1, [s1, q1, _row(g1), _row(be1),
                           W2.T, _row(b2)], 128, 64)
    z3, s3, q3 = _mid(z2, [s2, q2, _row(g2), _row(be2),
                           W3.T, _row(b3)], 64, 32)
    msg = _final_call(z3, [s3, q3, _row(g3), _row(be3),
                           Wf.T, _row(bf)], x_src)

    zeros = jnp.zeros((N, F_OUT), jnp.float32)
    ones = jnp.ones((SCCH, F_OUT), jnp.float32)
    part, degp = _sc_scatter(msg, dst, zeros, ones)

    p2 = part.reshape(NC, N // 8, 128)
    d2 = degp.reshape(NC, N // 8, 128)
    bias8 = jnp.tile(bias, 8).reshape(1, 128)
    out = _epilogue(p2, d2, bias8)
    return out.reshape(N, F_OUT)


# speedup vs baseline: 1.2870x; 1.0010x over previous
import functools

import jax
import jax.numpy as jnp
from jax import lax
from jax.experimental import pallas as pl
from jax.experimental.pallas import tpu as pltpu
from jax.experimental.pallas import tpu_sc as plsc

N = 10000
E = 320000
F_IN = 16
F_OUT = 16
EPS = 1e-5

TE = 8000
GRID = E // TE
TEB = 16000
GRIDB = E // TEB

NC = 2
NS = 16
NW = NC * NS
PER_W = E // NW
SCCH = 2000
NCHUNK = PER_W // SCCH

_ARB = pltpu.CompilerParams(dimension_semantics=("arbitrary",))


def _colsum(z):
    ones_row = jnp.ones((1, z.shape[0]), jnp.float32)
    return jnp.dot(ones_row, z, preferred_element_type=jnp.float32)


def _acc_stats(i, z, s_ref, q_ref):
    ps = _colsum(z)
    pq = _colsum(z * z)

    @pl.when(i == 0)
    def _():
        s_ref[...] = ps
        q_ref[...] = pq

    @pl.when(i > 0)
    def _():
        s_ref[...] = s_ref[...] + ps
        q_ref[...] = q_ref[...] + pq


def _bn_relu(z, s_ref, q_ref, g_ref, be_ref):
    m = s_ref[...] * (1.0 / E)
    v = q_ref[...] * (1.0 / E) - m * m
    a = g_ref[...] * lax.rsqrt(v + EPS)
    c = be_ref[...] - m * a
    return jnp.maximum(z * a + c, 0.0)


def _lin(h, wt_ref, b_ref):
    return jnp.dot(h, wt_ref[...], preferred_element_type=jnp.float32) + b_ref[...]


def _lin16(h, wt_ref, b_ref):
    return jnp.dot(h.astype(jnp.bfloat16), wt_ref[...],
                   preferred_element_type=jnp.float32) + b_ref[...]


def _h0_of(ef_ref, w0t_ref, b0_ref, ms_ref, m2_ref, g0_ref, be0_ref):
    z0 = lax.dot_general(ef_ref[...], w0t_ref[...],
                         (((0,), (0,)), ((), ())),
                         preferred_element_type=jnp.float32) + b0_ref[...]
    inv_e = 1.0 / E
    mu = [ms_ref[j, 0] * inv_e for j in range(3)]
    wrow = [w0t_ref[j:j + 1, :] for j in range(3)]
    m0 = mu[0] * wrow[0] + mu[1] * wrow[1] + mu[2] * wrow[2] + b0_ref[...]
    var = jnp.zeros_like(m0)
    for j in range(3):
        for k in range(3):
            cjk = m2_ref[j, k] * inv_e - mu[j] * mu[k]
            var = var + cjk * (wrow[j] * wrow[k])
    a0 = g0_ref[...] * lax.rsqrt(var + EPS)
    c0 = be0_ref[...] - m0 * a0
    return jnp.maximum(z0 * a0 + c0, 0.0)


def _moments_body(f_ref, ms_ref, m2_ref):
    e = f_ref[...]
    ones_col = jnp.ones((1, E), jnp.float32)
    ms_ref[...] = lax.dot_general(e, ones_col, (((1,), (1,)), ((), ())),
                                  preferred_element_type=jnp.float32)
    m2_ref[...] = lax.dot_general(e, e, (((1,), (1,)), ((), ())),
                                  preferred_element_type=jnp.float32)


def _p1_body(ef_ref, w0t_ref, b0_ref, ms_ref, m2_ref, g0_ref, be0_ref,
             w1t_ref, b1_ref,
             z1_ref, s1_ref, q1_ref):
    i = pl.program_id(0)
    h0 = _h0_of(ef_ref, w0t_ref, b0_ref, ms_ref, m2_ref, g0_ref, be0_ref)
    z1 = _lin(h0, w1t_ref, b1_ref)
    z1_ref[...] = z1.astype(jnp.bfloat16)
    _acc_stats(i, z1, s1_ref, q1_ref)


def _mid_body(z_ref, s_ref, q_ref, g_ref, be_ref, wt_ref, b_ref,
              zo_ref, so_ref, qo_ref):
    i = pl.program_id(0)
    h = _bn_relu(z_ref[...].astype(jnp.float32), s_ref, q_ref, g_ref, be_ref)
    z = _lin(h, wt_ref, b_ref)
    zo_ref[...] = z.astype(jnp.bfloat16)
    _acc_stats(i, z, so_ref, qo_ref)


def _p4_body(z3_ref, s3_ref, q3_ref, g3_ref, be3_ref,
             wft_ref, bf_ref, x_ref,
             msg_ref):
    h3 = _bn_relu(z3_ref[...].astype(jnp.float32),
                  s3_ref, q3_ref, g3_ref, be3_ref)
    wv = jax.nn.sigmoid(_lin(h3, wft_ref, bf_ref))
    x = x_ref[...]
    li = lax.broadcasted_iota(jnp.int32, (F_IN, F_IN * F_OUT), 1)
    ri = lax.broadcasted_iota(jnp.int32, (F_IN, F_IN * F_OUT), 0)
    rmat = (li // F_OUT == ri).astype(jnp.float32)
    si = lax.broadcasted_iota(jnp.int32, (F_IN * F_OUT, F_OUT), 0)
    oi = lax.broadcasted_iota(jnp.int32, (F_IN * F_OUT, F_OUT), 1)
    smat = (si % F_OUT == oi).astype(jnp.float32)
    xrep = jnp.dot(x, rmat, preferred_element_type=jnp.float32)
    msg_ref[...] = jnp.dot(xrep * wv, smat,
                           preferred_element_type=jnp.float32)


def _epi_body(p_ref, d_ref, bias_ref, out_ref):
    p = p_ref[...]
    d = d_ref[...]
    deg = jnp.maximum(d[0] + d[1], 1.0)
    out_ref[...] = (p[0] + p[1]) / deg + bias_ref[...]


def _row(x):
    return x.reshape(1, -1)


def _const_spec(x):
    shape = x.shape
    return pl.BlockSpec(shape, lambda i: (0,) * len(shape))


_EF_SPEC = pl.BlockSpec((TE, 3), lambda i: (i, 0))


def _moments(ef_t):
    return pl.pallas_call(
        _moments_body,
        out_shape=[jax.ShapeDtypeStruct((3, 1), jnp.float32),
                   jax.ShapeDtypeStruct((3, 3), jnp.float32)],
    )(ef_t)


def _layer1(ef_t, consts):
    return pl.pallas_call(
        _p1_body,
        grid=(GRIDB,),
        in_specs=[pl.BlockSpec((3, TEB), lambda i: (0, i))]
        + [_const_spec(c) for c in consts],
        out_specs=[pl.BlockSpec((TEB, 128), lambda i: (i, 0)),
                   pl.BlockSpec((1, 128), lambda i: (0, 0)),
                   pl.BlockSpec((1, 128), lambda i: (0, 0))],
        out_shape=[jax.ShapeDtypeStruct((E, 128), jnp.bfloat16),
                   jax.ShapeDtypeStruct((1, 128), jnp.float32),
                   jax.ShapeDtypeStruct((1, 128), jnp.float32)],
        compiler_params=_ARB,
    )(ef_t, *consts)


def _mid(z, consts, cin, cout):
    return pl.pallas_call(
        _mid_body,
        grid=(GRIDB,),
        in_specs=[pl.BlockSpec((TEB, cin), lambda i: (i, 0))]
        + [_const_spec(c) for c in consts],
        out_specs=[pl.BlockSpec((TEB, cout), lambda i: (i, 0)),
                   pl.BlockSpec((1, cout), lambda i: (0, 0)),
                   pl.BlockSpec((1, cout), lambda i: (0, 0))],
        out_shape=[jax.ShapeDtypeStruct((E, cout), jnp.bfloat16),
                   jax.ShapeDtypeStruct((1, cout), jnp.float32),
                   jax.ShapeDtypeStruct((1, cout), jnp.float32)],
        compiler_params=_ARB,
    )(z, *consts)


def _final_call(z3, consts, x_src):
    return pl.pallas_call(
        _p4_body,
        grid=(GRID,),
        in_specs=[pl.BlockSpec((TE, 32), lambda i: (i, 0))]
        + [_const_spec(c) for c in consts]
        + [pl.BlockSpec((TE, F_IN), lambda i: (i, 0))],
        out_specs=pl.BlockSpec((TE, F_OUT), lambda i: (i, 0)),
        out_shape=jax.ShapeDtypeStruct((E, F_OUT), jnp.float32),
        compiler_params=_ARB,
    )(z3, *consts, x_src)


def _epilogue(p2, d2, bias8):
    return pl.pallas_call(
        _epi_body,
        out_shape=jax.ShapeDtypeStruct((N // 8, 128), jnp.float32),
    )(p2, d2, bias8)


@functools.lru_cache(maxsize=None)
def _sc_kernels():
    mesh = plsc.VectorSubcoreMesh(core_axis_name="c", subcore_axis_name="s")

    @functools.partial(
        pl.kernel,
        mesh=mesh,
        out_type=jax.ShapeDtypeStruct((E, F_IN), jnp.float32),
        scratch_types=[pltpu.VMEM((SCCH,), jnp.int32),
                       pltpu.VMEM((SCCH, F_IN), jnp.float32),
                       pltpu.SemaphoreType.DMA],
        compiler_params=pltpu.CompilerParams(use_tc_tiling_on_sc=False),
    )
    def gather(nfeat_hbm, src_hbm, out_hbm, idx_v, rows_v, sem):
        wid = lax.axis_index("s") * NC + lax.axis_index("c")
        base = wid * PER_W

        def body(j, carry):
            off = base + j * SCCH
            pltpu.sync_copy(src_hbm.at[pl.ds(off, SCCH)], idx_v)
            pltpu.async_copy(nfeat_hbm.at[idx_v], rows_v, sem).wait()
            pltpu.sync_copy(rows_v, out_hbm.at[pl.ds(off, SCCH)])
            return carry

        lax.fori_loop(0, NCHUNK, body, 0)

    @functools.partial(
        pl.kernel,
        mesh=mesh,
        out_type=[jax.ShapeDtypeStruct((NC, N, F_OUT), jnp.float32),
                  jax.ShapeDtypeStruct((NC, N, F_OUT), jnp.float32)],
        scratch_types=[pltpu.VMEM((SCCH,), jnp.int32),
                       pltpu.VMEM((SCCH, F_OUT), jnp.float32),
                       pltpu.VMEM((SCCH, F_OUT), jnp.float32),
                       pltpu.VMEM_SHARED((N, F_OUT), jnp.float32),
                       pltpu.VMEM_SHARED((N, F_OUT), jnp.float32)],
        compiler_params=pltpu.CompilerParams(use_tc_tiling_on_sc=False),
    )
    def scatter(msg_hbm, dst_hbm, zeros_hbm, ones_hbm,
                neigh_out, deg_out, idx_v, m_v, ones_v, acc_sh, deg_sh):
        c = lax.axis_index("c")
        s = lax.axis_index("s")
        wid = s * NC + c
        base = wid * PER_W

        @pl.when(s == 0)
        def _():
            pltpu.sync_copy(zeros_hbm, acc_sh)
            pltpu.sync_copy(zeros_hbm, deg_sh)

        pltpu.sync_copy(ones_hbm, ones_v)
        plsc.subcore_barrier()

        def body(j, carry):
            off = base + j * SCCH
            pltpu.sync_copy(dst_hbm.at[pl.ds(off, SCCH)], idx_v)
            pltpu.sync_copy(msg_hbm.at[pl.ds(off, SCCH)], m_v)
            pltpu.sync_copy(m_v, acc_sh.at[idx_v], add=True)
            pltpu.sync_copy(ones_v, deg_sh.at[idx_v], add=True)
            return carry

        lax.fori_loop(0, NCHUNK, body, 0)
        plsc.subcore_barrier()

        @pl.when(s == 0)
        def _():
            pltpu.sync_copy(acc_sh, neigh_out.at[c])
            pltpu.sync_copy(deg_sh, deg_out.at[c])

    return gather, scatter


def _sc_gather(nfeat, src):
    return _sc_kernels()[0](nfeat, src)


def _sc_scatter(msg, dst, zeros, ones):
    return _sc_kernels()[1](msg, dst, zeros, ones)


def kernel(nfeat, edge_index, efeat,
           W0, b0, g0, be0,
           W1, b1, g1, be1,
           W2, b2, g2, be2,
           W3, b3, g3, be3,
           Wf, bf, bias):
    src = edge_index[0]
    dst = edge_index[1]

    ef_t = lax.optimization_barrier(efeat.T)
    ms, m2 = _moments(ef_t)
    x_src = _sc_gather(nfeat, src)

    c1 = [W0.T, _row(b0), ms, m2, _row(g0), _row(be0),
          W1.T, _row(b1)]
    z1, s1, q1 = _layer1(ef_t, c1)
    z2, s2, q2 = _mid(z1, [s1, q1, _row(g1), _row(be1),
                           W2.T, _row(b2)], 128, 64)
    z3, s3, q3 = _mid(z2, [s2, q2, _row(g2), _row(be2),
                           W3.T, _row(b3)], 64, 32)
    msg = _final_call(z3, [s3, q3, _row(g3), _row(be3),
                           Wf.T, _row(bf)], x_src)

    zeros = jnp.zeros((N, F_OUT), jnp.float32)
    ones = jnp.ones((SCCH, F_OUT), jnp.float32)
    part, degp = _sc_scatter(msg, dst, zeros, ones)

    p2 = part.reshape(NC, N // 8, 128)
    d2 = degp.reshape(NC, N // 8, 128)
    bias8 = jnp.tile(bias, 8).reshape(1, 128)
    out = _epilogue(p2, d2, bias8)
    return out.reshape(N, F_OUT)
